# Initial kernel scaffold; baseline (speedup 1.0000x reference)
import jax
import jax.numpy as jnp
from jax.experimental import pallas as pl


def kernel(x, conv_w, conv_b, bn_gamma, bn_beta):
    raise NotImplementedError("write your pallas kernel here")


# speedup vs baseline: 3.5185x; 3.5185x over previous
import functools

import jax
import jax.numpy as jnp
from jax import lax
from jax.experimental import pallas as pl
from jax.experimental.pallas import tpu as pltpu

_BN_EPS = 1e-5
_VMEM_LIMIT = 64 * 1024 * 1024


def _round_up(v, m):
    return (v + m - 1) // m * m


def _conv_stats_kernel(x_ref, w_ref, y_ref, sum_ref, sq_ref, *, taps, flat_len,
                       wp, w_valid, h_valid):
    p = jnp.concatenate([x_ref[0, :, off:off + flat_len] for off in taps],
                        axis=0)
    y = jnp.dot(w_ref[...], p, preferred_element_type=jnp.float32)
    col = lax.broadcasted_iota(jnp.int32, y.shape, 1)
    valid = (col < h_valid * wp) & (col % wp < w_valid)
    ym = jnp.where(valid, y, 0.0)
    sum_ref[0] = jnp.sum(ym, axis=1, keepdims=True)
    sq_ref[0] = jnp.sum(ym * ym, axis=1, keepdims=True)
    y_ref[0] = ym.astype(y_ref.dtype)


def _bn_sigmoid_kernel(y_ref, scale_ref, shift_ref, o_ref, *, w_valid):
    c_out = o_ref.shape[1]
    z = y_ref[0, :, :, :w_valid].astype(jnp.float32)
    z = z * scale_ref[...].reshape(c_out, 1, 1) + shift_ref[...].reshape(
        c_out, 1, 1)
    o_ref[0] = pl.reciprocal(1.0 + jnp.exp(-z), approx=False)


@jax.jit
def kernel(x, conv_w, conv_b, bn_gamma, bn_beta):
    del conv_b
    n, c_in, h, w = x.shape
    c_out, _, kh, kw = conv_w.shape
    pad = 1
    hp, wp = h + 2 * pad, w + 2 * pad
    flat_len = hp * wp
    taps = tuple(dy * wp + dx for dy in range(kh) for dx in range(kw))
    xcols = _round_up(flat_len + taps[-1], 128)

    xf = jnp.pad(x, ((0, 0), (0, 0), (pad, pad), (pad, pad)))
    xf = xf.astype(jnp.bfloat16).reshape(n, c_in, flat_len)
    xf = jnp.pad(xf, ((0, 0), (0, 0), (0, xcols - flat_len)))
    wmat = conv_w.transpose(0, 2, 3, 1).reshape(c_out, kh * kw * c_in)
    wmat = wmat.astype(jnp.bfloat16)

    y, psum, psq = pl.pallas_call(
        functools.partial(_conv_stats_kernel, taps=taps, flat_len=flat_len,
                          wp=wp, w_valid=w, h_valid=h),
        out_shape=(
            jax.ShapeDtypeStruct((n, c_out, flat_len), jnp.bfloat16),
            jax.ShapeDtypeStruct((n, c_out, 1), jnp.float32),
            jax.ShapeDtypeStruct((n, c_out, 1), jnp.float32),
        ),
        grid=(n,),
        in_specs=[
            pl.BlockSpec((1, c_in, xcols), lambda i: (i, 0, 0)),
            pl.BlockSpec((c_out, kh * kw * c_in), lambda i: (0, 0)),
        ],
        out_specs=(
            pl.BlockSpec((1, c_out, flat_len), lambda i: (i, 0, 0)),
            pl.BlockSpec((1, c_out, 1), lambda i: (i, 0, 0)),
            pl.BlockSpec((1, c_out, 1), lambda i: (i, 0, 0)),
        ),
        compiler_params=pltpu.CompilerParams(
            dimension_semantics=("parallel",),
            vmem_limit_bytes=_VMEM_LIMIT),
    )(xf, wmat)

    inv_m = 1.0 / float(n * h * w)
    sum_y = jnp.sum(psum, axis=0)
    sum_y2 = jnp.sum(psq, axis=0)
    mean = sum_y * inv_m
    var = jnp.maximum(sum_y2 * inv_m - mean * mean, 0.0)
    scale = bn_gamma.reshape(c_out, 1) * lax.rsqrt(var + _BN_EPS)
    shift = bn_beta.reshape(c_out, 1) - mean * scale

    y4 = y.reshape(n, c_out, hp, wp)
    out = pl.pallas_call(
        functools.partial(_bn_sigmoid_kernel, w_valid=w),
        out_shape=jax.ShapeDtypeStruct((n, c_out, h, w), jnp.float32),
        grid=(n,),
        in_specs=[
            pl.BlockSpec((1, c_out, h, wp), lambda i: (i, 0, 0, 0)),
            pl.BlockSpec((c_out, 1), lambda i: (0, 0)),
            pl.BlockSpec((c_out, 1), lambda i: (0, 0)),
        ],
        out_specs=pl.BlockSpec((1, c_out, h, w), lambda i: (i, 0, 0, 0)),
        compiler_params=pltpu.CompilerParams(
            dimension_semantics=("parallel",),
            vmem_limit_bytes=_VMEM_LIMIT),
    )(y4, scale, shift)
    return out


# speedup vs baseline: 3.6952x; 1.0502x over previous
import functools

import jax
import jax.numpy as jnp
from jax import lax
from jax.experimental import pallas as pl
from jax.experimental.pallas import tpu as pltpu

_BN_EPS = 1e-5
_VMEM_LIMIT = 64 * 1024 * 1024


def _round_up(v, m):
    return (v + m - 1) // m * m


def _conv_stats_kernel(x_ref, w_ref, y_ref, sum_ref, sq_ref, *, taps, flat_len,
                       wp, w_valid):
    p = jnp.concatenate([x_ref[0, :, off:off + flat_len] for off in taps],
                        axis=0)
    y = jnp.dot(w_ref[...], p, preferred_element_type=jnp.float32)
    col = lax.broadcasted_iota(jnp.int32, y.shape, 1)
    ym = jnp.where(col % wp < w_valid, y, 0.0)
    sum_ref[0] = jnp.sum(ym, axis=1, keepdims=True)
    sq_ref[0] = jnp.sum(ym * ym, axis=1, keepdims=True)
    y_ref[0] = ym.astype(y_ref.dtype)


def _bn_sigmoid_kernel(y_ref, scale_ref, shift_ref, o_ref, *, w_valid):
    c_out = o_ref.shape[1]
    z = y_ref[0, :, :, :w_valid].astype(jnp.float32)
    z = z * scale_ref[...].reshape(c_out, 1, 1) + shift_ref[...].reshape(
        c_out, 1, 1)
    o_ref[0] = pl.reciprocal(1.0 + jnp.exp(-z), approx=False)


@jax.jit
def kernel(x, conv_w, conv_b, bn_gamma, bn_beta):
    del conv_b
    n, c_in, h, w = x.shape
    c_out, _, kh, kw = conv_w.shape
    pad = 1
    hp, wp = h + 3 * pad, w + 5 * pad
    flat_len = h * wp
    taps = tuple(dy * wp + dx for dy in range(kh) for dx in range(kw))
    assert hp * wp >= flat_len + taps[-1]

    xf = jnp.pad(x, ((0, 0), (0, 0), (pad, 2 * pad), (pad, 4 * pad)))
    xf = xf.astype(jnp.bfloat16).reshape(n, c_in, hp * wp)
    wmat = conv_w.transpose(0, 2, 3, 1).reshape(c_out, kh * kw * c_in)
    wmat = wmat.astype(jnp.bfloat16)

    y, psum, psq = pl.pallas_call(
        functools.partial(_conv_stats_kernel, taps=taps, flat_len=flat_len,
                          wp=wp, w_valid=w),
        out_shape=(
            jax.ShapeDtypeStruct((n, c_out, flat_len), jnp.bfloat16),
            jax.ShapeDtypeStruct((n, c_out, 1), jnp.float32),
            jax.ShapeDtypeStruct((n, c_out, 1), jnp.float32),
        ),
        grid=(n,),
        in_specs=[
            pl.BlockSpec((1, c_in, hp * wp), lambda i: (i, 0, 0)),
            pl.BlockSpec((c_out, kh * kw * c_in), lambda i: (0, 0)),
        ],
        out_specs=(
            pl.BlockSpec((1, c_out, flat_len), lambda i: (i, 0, 0)),
            pl.BlockSpec((1, c_out, 1), lambda i: (i, 0, 0)),
            pl.BlockSpec((1, c_out, 1), lambda i: (i, 0, 0)),
        ),
        compiler_params=pltpu.CompilerParams(
            dimension_semantics=("parallel",),
            vmem_limit_bytes=_VMEM_LIMIT),
    )(xf, wmat)

    inv_m = 1.0 / float(n * h * w)
    sum_y = jnp.sum(psum, axis=0)
    sum_y2 = jnp.sum(psq, axis=0)
    mean = sum_y * inv_m
    var = jnp.maximum(sum_y2 * inv_m - mean * mean, 0.0)
    scale = bn_gamma.reshape(c_out, 1) * lax.rsqrt(var + _BN_EPS)
    shift = bn_beta.reshape(c_out, 1) - mean * scale

    y4 = y.reshape(n, c_out, h, wp)
    out = pl.pallas_call(
        functools.partial(_bn_sigmoid_kernel, w_valid=w),
        out_shape=jax.ShapeDtypeStruct((n, c_out, h, w), jnp.float32),
        grid=(n,),
        in_specs=[
            pl.BlockSpec((1, c_out, h, wp), lambda i: (i, 0, 0, 0)),
            pl.BlockSpec((c_out, 1), lambda i: (0, 0)),
            pl.BlockSpec((c_out, 1), lambda i: (0, 0)),
        ],
        out_specs=pl.BlockSpec((1, c_out, h, w), lambda i: (i, 0, 0, 0)),
        compiler_params=pltpu.CompilerParams(
            dimension_semantics=("parallel",),
            vmem_limit_bytes=_VMEM_LIMIT),
    )(y4, scale, shift)
    return out


# speedup vs baseline: 4.3629x; 1.1807x over previous
import functools

import jax
import jax.numpy as jnp
from jax import lax
from jax.experimental import pallas as pl
from jax.experimental.pallas import tpu as pltpu

_BN_EPS = 1e-5
_VMEM_LIMIT = 64 * 1024 * 1024


def _flatten_pad(x3, pad):
    xq = x3.astype(jnp.bfloat16)
    xp = jnp.pad(xq, ((0, 0), (pad, 2 * pad), (pad, 4 * pad)))
    return xp.reshape(x3.shape[0], -1)


def _patches(xf, taps, flat_len):
    return jnp.concatenate([xf[:, off:off + flat_len] for off in taps], axis=0)


def _stats_kernel(x_ref, w_ref, sum_ref, sq_ref, *, pad, taps, flat_len, wp,
                  w_valid):
    xf = _flatten_pad(x_ref[0], pad)
    y = jnp.dot(w_ref[...], _patches(xf, taps, flat_len),
                preferred_element_type=jnp.float32)
    col = lax.broadcasted_iota(jnp.int32, y.shape, 1)
    ym = jnp.where(col % wp < w_valid, y, 0.0)
    sum_ref[0] = jnp.sum(ym, axis=1, keepdims=True)
    sq_ref[0] = jnp.sum(ym * ym, axis=1, keepdims=True)


def _conv_bn_sigmoid_kernel(x_ref, w_ref, scale_ref, shift_ref, o_ref, *, pad,
                            taps, flat_len, wp, w_valid, h_valid):
    c_out = o_ref.shape[1]
    xf = _flatten_pad(x_ref[0], pad)
    y = jnp.dot(w_ref[...], _patches(xf, taps, flat_len),
                preferred_element_type=jnp.float32)
    z = y * scale_ref[...].reshape(c_out, 1) + shift_ref[...].reshape(c_out, 1)
    s = pl.reciprocal(1.0 + jnp.exp(-z), approx=False)
    o_ref[0] = s.reshape(c_out, h_valid, wp)[:, :, :w_valid]


@jax.jit
def kernel(x, conv_w, conv_b, bn_gamma, bn_beta):
    del conv_b
    n, c_in, h, w = x.shape
    c_out, _, kh, kw = conv_w.shape
    pad = 1
    hp, wp = h + 3 * pad, w + 5 * pad
    flat_len = h * wp
    taps = tuple(dy * wp + dx for dy in range(kh) for dx in range(kw))
    assert hp * wp >= flat_len + taps[-1]

    wmat = conv_w.transpose(0, 2, 3, 1).reshape(c_out, kh * kw * c_in)
    wmat = wmat.astype(jnp.bfloat16)

    kw_common = dict(pad=pad, taps=taps, flat_len=flat_len, wp=wp, w_valid=w)
    psum, psq = pl.pallas_call(
        functools.partial(_stats_kernel, **kw_common),
        out_shape=(
            jax.ShapeDtypeStruct((n, c_out, 1), jnp.float32),
            jax.ShapeDtypeStruct((n, c_out, 1), jnp.float32),
        ),
        grid=(n,),
        in_specs=[
            pl.BlockSpec((1, c_in, h, w), lambda i: (i, 0, 0, 0)),
            pl.BlockSpec((c_out, kh * kw * c_in), lambda i: (0, 0)),
        ],
        out_specs=(
            pl.BlockSpec((1, c_out, 1), lambda i: (i, 0, 0)),
            pl.BlockSpec((1, c_out, 1), lambda i: (i, 0, 0)),
        ),
        compiler_params=pltpu.CompilerParams(
            dimension_semantics=("parallel",),
            vmem_limit_bytes=_VMEM_LIMIT),
    )(x, wmat)

    inv_m = 1.0 / float(n * h * w)
    sum_y = jnp.sum(psum, axis=0)
    sum_y2 = jnp.sum(psq, axis=0)
    mean = sum_y * inv_m
    var = jnp.maximum(sum_y2 * inv_m - mean * mean, 0.0)
    scale = bn_gamma.reshape(c_out, 1) * lax.rsqrt(var + _BN_EPS)
    shift = bn_beta.reshape(c_out, 1) - mean * scale

    out = pl.pallas_call(
        functools.partial(_conv_bn_sigmoid_kernel, **kw_common, h_valid=h),
        out_shape=jax.ShapeDtypeStruct((n, c_out, h, w), jnp.float32),
        grid=(n,),
        in_specs=[
            pl.BlockSpec((1, c_in, h, w), lambda i: (i, 0, 0, 0)),
            pl.BlockSpec((c_out, kh * kw * c_in), lambda i: (0, 0)),
            pl.BlockSpec((c_out, 1), lambda i: (0, 0)),
            pl.BlockSpec((c_out, 1), lambda i: (0, 0)),
        ],
        out_specs=pl.BlockSpec((1, c_out, h, w), lambda i: (i, 0, 0, 0)),
        compiler_params=pltpu.CompilerParams(
            dimension_semantics=("parallel",),
            vmem_limit_bytes=_VMEM_LIMIT),
    )(x, wmat, scale, shift)
    return out
